# baseline (speedup 1.0000x reference)
import functools

import jax
import jax.numpy as jnp
from jax import lax
from jax.experimental import pallas as pl
from jax.experimental.pallas import tpu as pltpu
from jax.experimental.pallas import tpu_sc as plsc

N = 10000
E = 320000
D_H = 128
D_OUT = 16

NC = 2
NS = 16
NW = NC * NS
CHUNK = 128
NCHUNK = 80
NCH0 = 128
NCH1 = 32
TOTCH = NS * (NCH0 + NCH1)
IDXBUF = 8
NGROUP = NCHUNK // IDXBUF
E_PAD = NW * NCHUNK * CHUNK
N_PAD = 10240
ROWS = N_PAD // NS
DEG_W = 16
BR = 400


@functools.lru_cache(None)
def _seg_sum_kernel():
    mesh = plsc.VectorSubcoreMesh(core_axis_name="c", subcore_axis_name="s")

    def body(h_hbm, src_hbm, dst_hbm, zeros_hbm, agg_hbm,
             srcv, dstv, bufa, bufb, acc, sema, semb):
        c = lax.axis_index("c")
        s = lax.axis_index("s")
        r0 = s * ROWS
        nch = NCH0 - c * (NCH0 - NCH1)
        base = s * NCH0 + c * (NS * NCH0 - (NCH0 - NCH1) * s)
        pltpu.sync_copy(zeros_hbm.at[pl.ds(0, CHUNK)], bufa)
        for r in range(ROWS // CHUNK):
            pltpu.sync_copy(bufa, acc.at[pl.ds(r0 + r * CHUNK, CHUNK)])
        plsc.subcore_barrier()

        bufs = (bufa, bufb)
        sems = (sema, semb)

        @pl.loop(0, nch // IDXBUF)
        def _(g):
            g0 = pl.multiple_of(base + g * IDXBUF, IDXBUF)
            pltpu.sync_copy(src_hbm.at[pl.ds(g0, IDXBUF)], srcv)
            pltpu.sync_copy(dst_hbm.at[pl.ds(g0, IDXBUF)], dstv)
            cps = [pltpu.async_copy(h_hbm.at[srcv.at[0]], bufs[0], sems[0])]
            for jj in range(IDXBUF):
                if jj + 1 < IDXBUF:
                    cps.append(pltpu.async_copy(
                        h_hbm.at[srcv.at[jj + 1]],
                        bufs[(jj + 1) % 2], sems[(jj + 1) % 2]))
                cps[jj].wait()
                pltpu.sync_copy(bufs[jj % 2], acc.at[dstv.at[jj]], add=True)

        plsc.subcore_barrier()
        for r in range(ROWS // CHUNK):
            rr = r0 + r * CHUNK
            pltpu.sync_copy(acc.at[pl.ds(rr, CHUNK)], bufa)
            pltpu.sync_copy(bufa, agg_hbm.at[c, pl.ds(rr, CHUNK)])

    return pl.kernel(
        body, mesh=mesh,
        out_type=jax.ShapeDtypeStruct((NC, N_PAD, D_H), jnp.float32),
        scratch_types=[
            pltpu.VMEM((IDXBUF, CHUNK), jnp.int32),
            pltpu.VMEM((IDXBUF, CHUNK), jnp.int32),
            pltpu.VMEM((CHUNK, D_H), jnp.float32),
            pltpu.VMEM((CHUNK, D_H), jnp.float32),
            pltpu.VMEM_SHARED((N_PAD, D_H), jnp.float32),
            pltpu.SemaphoreType.DMA,
            pltpu.SemaphoreType.DMA,
        ])


@functools.lru_cache(None)
def _deg_kernel():
    mesh = plsc.VectorSubcoreMesh(core_axis_name="c", subcore_axis_name="s")

    def body(dst_hbm, zeros_hbm, ones_hbm, deg_hbm, dstv, buf, onesv, acc):
        c = lax.axis_index("c")
        s = lax.axis_index("s")
        wid = s * NC + c
        r0 = s * ROWS
        pltpu.sync_copy(zeros_hbm.at[pl.ds(0, CHUNK)], buf)
        for r in range(ROWS // CHUNK):
            pltpu.sync_copy(buf, acc.at[pl.ds(r0 + r * CHUNK, CHUNK)])
        pltpu.sync_copy(ones_hbm, onesv)
        plsc.subcore_barrier()

        for g in range(NGROUP):
            gd = pl.multiple_of(wid * NCHUNK + g * IDXBUF, IDXBUF)
            pltpu.sync_copy(dst_hbm.at[pl.ds(gd, IDXBUF)], dstv)

            @pl.loop(0, IDXBUF)
            def _(j):
                pltpu.sync_copy(onesv, acc.at[dstv.at[j]], add=True)

        plsc.subcore_barrier()
        for r in range(ROWS // CHUNK):
            rr = r0 + r * CHUNK
            pltpu.sync_copy(acc.at[pl.ds(rr, CHUNK)], buf)
            pltpu.sync_copy(buf, deg_hbm.at[c, pl.ds(rr, CHUNK)])

    return pl.kernel(
        body, mesh=mesh,
        out_type=jax.ShapeDtypeStruct((NC, N_PAD, D_H), jnp.float32),
        scratch_types=[
            pltpu.VMEM((IDXBUF, CHUNK), jnp.int32),
            pltpu.VMEM((CHUNK, D_H), jnp.float32),
            pltpu.VMEM((CHUNK, D_H), jnp.float32),
            pltpu.VMEM_SHARED((N_PAD, D_H), jnp.float32),
        ])


def _mean(agg_blk, deg_blk):
    deg = deg_blk[0, :, 0:1] + deg_blk[1, :, 0:1]
    inv = 1.0 / jnp.maximum(deg, 1.0)
    return (agg_blk[0] + agg_blk[1]) * inv


def _layer_tc(h, agg, deg, Wself, Wneigh, b, Wnext=None):
    dn = 0 if Wnext is None else Wnext.shape[1]

    def kfn(h_ref, agg_ref, deg_ref, ws_ref, wn_ref, b_ref, *rest):
        if Wnext is None:
            o_ref, = rest
        else:
            wn2_ref, o_ref, o2_ref = rest
        mean = _mean(agg_ref[...], deg_ref[...])
        acc = jnp.dot(h_ref[...], ws_ref[...],
                      preferred_element_type=jnp.float32)
        acc = acc + jnp.dot(mean, wn_ref[...],
                            preferred_element_type=jnp.float32)
        hn = jnp.maximum(acc + b_ref[...], 0.0)
        o_ref[...] = hn
        if Wnext is not None:
            o2_ref[...] = jnp.dot(hn, wn2_ref[...],
                                  preferred_element_type=jnp.float32)

    in_specs = [
        pl.BlockSpec((BR, D_H), lambda i: (i, 0)),
        pl.BlockSpec((NC, BR, D_H), lambda i: (0, i, 0)),
        pl.BlockSpec((NC, BR, D_H), lambda i: (0, i, 0)),
        pl.BlockSpec((D_H, D_H), lambda i: (0, 0)),
        pl.BlockSpec((D_H, D_H), lambda i: (0, 0)),
        pl.BlockSpec((1, D_H), lambda i: (0, 0)),
    ]
    out_specs = [pl.BlockSpec((BR, D_H), lambda i: (i, 0))]
    out_shape = [jax.ShapeDtypeStruct((N, D_H), jnp.float32)]
    args = [h, agg, deg, Wself, Wneigh, b.reshape(1, D_H)]
    if Wnext is not None:
        in_specs.append(pl.BlockSpec((D_H, dn), lambda i: (0, 0)))
        out_specs.append(pl.BlockSpec((BR, dn), lambda i: (i, 0)))
        out_shape.append(jax.ShapeDtypeStruct((N, dn), jnp.float32))
        args.append(Wnext)
    res = pl.pallas_call(
        kfn, grid=(N // BR,), in_specs=in_specs, out_specs=out_specs,
        out_shape=out_shape)(*args)
    return res[0] if Wnext is None else (res[0], res[1])


def _final_tc(h, agg, deg, Wself, Wneigh, b):
    def kfn(h_ref, agg_ref, deg_ref, ws_ref, wn_ref, b_ref, o_ref):
        mean = _mean(agg_ref[...], deg_ref[...])
        o_ref[...] = jnp.dot(h_ref[...], ws_ref[...],
                             preferred_element_type=jnp.float32) \
            + jnp.dot(mean, wn_ref[...],
                      preferred_element_type=jnp.float32) \
            + b_ref[...]

    return pl.pallas_call(
        kfn, grid=(N // BR,),
        in_specs=[
            pl.BlockSpec((BR, D_H), lambda i: (i, 0)),
            pl.BlockSpec((NC, BR, D_H), lambda i: (0, i, 0)),
            pl.BlockSpec((NC, BR, D_H), lambda i: (0, i, 0)),
            pl.BlockSpec((D_H, D_OUT), lambda i: (0, 0)),
            pl.BlockSpec((D_H, D_OUT), lambda i: (0, 0)),
            pl.BlockSpec((1, D_OUT), lambda i: (0, 0)),
        ],
        out_specs=pl.BlockSpec((BR, D_OUT), lambda i: (i, 0)),
        out_shape=jax.ShapeDtypeStruct((N, D_OUT), jnp.float32),
    )(h, agg, deg, Wself, Wneigh, b.reshape(1, D_OUT))


def kernel(x, edge_index, Wself0, Wneigh0, b0, Wself1, Wneigh1, b1,
           Wself2, Wneigh2, b2):
    src = edge_index[0]
    dst = edge_index[1]
    pad = E_PAD - E
    src3 = jnp.concatenate(
        [src, jnp.zeros((pad,), jnp.int32)]).reshape(TOTCH, CHUNK)
    dst3 = jnp.concatenate(
        [dst, jnp.full((pad,), N, jnp.int32)]).reshape(TOTCH, CHUNK)
    zeros128 = jnp.zeros((N_PAD, D_H), jnp.float32)
    ones128 = jnp.ones((CHUNK, D_H), jnp.float32)

    deg = _deg_kernel()(dst3, zeros128, ones128)
    agg0 = _seg_sum_kernel()(x, src3, dst3, zeros128)
    h1 = _layer_tc(x, agg0, deg, Wself0, Wneigh0, b0)
    agg1 = _seg_sum_kernel()(h1, src3, dst3, zeros128)
    h2 = _layer_tc(h1, agg1, deg, Wself1, Wneigh1, b1)
    agg2 = _seg_sum_kernel()(h2, src3, dst3, zeros128)
    return _final_tc(h2, agg2, deg, Wself2, Wneigh2, b2)


# speedup vs baseline: 1.0645x; 1.0645x over previous
import functools

import jax
import jax.numpy as jnp
from jax import lax
from jax.experimental import pallas as pl
from jax.experimental.pallas import tpu as pltpu
from jax.experimental.pallas import tpu_sc as plsc

N = 10000
E = 320000
D_H = 128
D_OUT = 16

NC = 2
NS = 16
NW = NC * NS
CHUNK = 128
NCHUNK = 80
IDXBUF = 8
NGROUP = NCHUNK // IDXBUF
E_PAD = NW * NCHUNK * CHUNK
N_PAD = 10240
ROWS = N_PAD // NS
DEG_W = 16
BR = 400


@functools.lru_cache(None)
def _seg_sum_kernel():
    mesh = plsc.VectorSubcoreMesh(core_axis_name="c", subcore_axis_name="s")

    def body(h_hbm, src_hbm, dst_hbm, zeros_hbm, agg_hbm,
             srcv, dstv, bufa, bufb, acc, sema, semb):
        c = lax.axis_index("c")
        hc_hbm = h_hbm.at[c]
        s = lax.axis_index("s")
        wid = s * NC + c
        r0 = s * ROWS
        pltpu.sync_copy(zeros_hbm.at[pl.ds(0, CHUNK)], bufa)
        for r in range(ROWS // CHUNK):
            pltpu.sync_copy(bufa, acc.at[pl.ds(r0 + r * CHUNK, CHUNK)])
        plsc.subcore_barrier()

        bufs = (bufa, bufb)
        sems = (sema, semb)

        @pl.loop(0, NGROUP)
        def _(g):
            g0 = pl.multiple_of(g * IDXBUF, IDXBUF)
            pltpu.sync_copy(src_hbm.at[wid, pl.ds(g0, IDXBUF)], srcv)
            pltpu.sync_copy(dst_hbm.at[wid, pl.ds(g0, IDXBUF)], dstv)
            cps = [pltpu.async_copy(hc_hbm.at[srcv.at[0]], bufs[0], sems[0])]
            for jj in range(IDXBUF):
                if jj + 1 < IDXBUF:
                    cps.append(pltpu.async_copy(
                        hc_hbm.at[srcv.at[jj + 1]],
                        bufs[(jj + 1) % 2], sems[(jj + 1) % 2]))
                cps[jj].wait()
                pltpu.sync_copy(bufs[jj % 2], acc.at[dstv.at[jj]], add=True)

        plsc.subcore_barrier()
        for r in range(ROWS // CHUNK):
            rr = r0 + r * CHUNK
            pltpu.sync_copy(acc.at[pl.ds(rr, CHUNK)], bufa)
            pltpu.sync_copy(bufa, agg_hbm.at[c, pl.ds(rr, CHUNK)])

    return pl.kernel(
        body, mesh=mesh,
        out_type=jax.ShapeDtypeStruct((NC, N_PAD, D_H), jnp.float32),
        scratch_types=[
            pltpu.VMEM((IDXBUF, CHUNK), jnp.int32),
            pltpu.VMEM((IDXBUF, CHUNK), jnp.int32),
            pltpu.VMEM((CHUNK, D_H), jnp.float32),
            pltpu.VMEM((CHUNK, D_H), jnp.float32),
            pltpu.VMEM_SHARED((N_PAD, D_H), jnp.float32),
            pltpu.SemaphoreType.DMA,
            pltpu.SemaphoreType.DMA,
        ])


@functools.lru_cache(None)
def _deg_kernel():
    mesh = plsc.VectorSubcoreMesh(core_axis_name="c", subcore_axis_name="s")

    def body(dst_hbm, zeros_hbm, ones_hbm, deg_hbm, dstv, buf, onesv, acc):
        c = lax.axis_index("c")
        s = lax.axis_index("s")
        wid = s * NC + c
        r0 = s * ROWS
        pltpu.sync_copy(zeros_hbm.at[pl.ds(0, CHUNK)], buf)
        for r in range(ROWS // CHUNK):
            pltpu.sync_copy(buf, acc.at[pl.ds(r0 + r * CHUNK, CHUNK)])
        pltpu.sync_copy(ones_hbm, onesv)
        plsc.subcore_barrier()

        for g in range(NGROUP):
            pltpu.sync_copy(dst_hbm.at[wid, pl.ds(g * IDXBUF, IDXBUF)], dstv)

            @pl.loop(0, IDXBUF)
            def _(j):
                pltpu.sync_copy(onesv, acc.at[dstv.at[j]], add=True)

        plsc.subcore_barrier()
        for r in range(ROWS // CHUNK):
            rr = r0 + r * CHUNK
            pltpu.sync_copy(acc.at[pl.ds(rr, CHUNK)], buf)
            pltpu.sync_copy(buf, deg_hbm.at[c, pl.ds(rr, CHUNK)])

    return pl.kernel(
        body, mesh=mesh,
        out_type=jax.ShapeDtypeStruct((NC, N_PAD, D_H), jnp.float32),
        scratch_types=[
            pltpu.VMEM((IDXBUF, CHUNK), jnp.int32),
            pltpu.VMEM((CHUNK, D_H), jnp.float32),
            pltpu.VMEM((CHUNK, D_H), jnp.float32),
            pltpu.VMEM_SHARED((N_PAD, D_H), jnp.float32),
        ])


def _mean(agg_blk, deg_blk):
    deg = deg_blk[0, :, 0:1] + deg_blk[1, :, 0:1]
    inv = 1.0 / jnp.maximum(deg, 1.0)
    return (agg_blk[0] + agg_blk[1]) * inv


def _layer_tc(h, agg, deg, Wself, Wneigh, b, Wnext=None):
    dn = 0 if Wnext is None else Wnext.shape[1]

    def kfn(h_ref, agg_ref, deg_ref, ws_ref, wn_ref, b_ref, *rest):
        if Wnext is None:
            o_ref, = rest
        else:
            wn2_ref, o_ref, o2_ref = rest
        mean = _mean(agg_ref[...], deg_ref[...])
        acc = jnp.dot(h_ref[...], ws_ref[...],
                      preferred_element_type=jnp.float32)
        acc = acc + jnp.dot(mean, wn_ref[...],
                            preferred_element_type=jnp.float32)
        hn = jnp.maximum(acc + b_ref[...], 0.0)
        o_ref[...] = hn
        if Wnext is not None:
            o2_ref[...] = jnp.dot(hn, wn2_ref[...],
                                  preferred_element_type=jnp.float32)

    in_specs = [
        pl.BlockSpec((BR, D_H), lambda i: (i, 0)),
        pl.BlockSpec((NC, BR, D_H), lambda i: (0, i, 0)),
        pl.BlockSpec((NC, BR, D_H), lambda i: (0, i, 0)),
        pl.BlockSpec((D_H, D_H), lambda i: (0, 0)),
        pl.BlockSpec((D_H, D_H), lambda i: (0, 0)),
        pl.BlockSpec((1, D_H), lambda i: (0, 0)),
    ]
    out_specs = [pl.BlockSpec((BR, D_H), lambda i: (i, 0))]
    out_shape = [jax.ShapeDtypeStruct((N, D_H), jnp.float32)]
    args = [h, agg, deg, Wself, Wneigh, b.reshape(1, D_H)]
    if Wnext is not None:
        in_specs.append(pl.BlockSpec((D_H, dn), lambda i: (0, 0)))
        out_specs.append(pl.BlockSpec((BR, dn), lambda i: (i, 0)))
        out_shape.append(jax.ShapeDtypeStruct((N, dn), jnp.float32))
        args.append(Wnext)
    res = pl.pallas_call(
        kfn, grid=(N // BR,), in_specs=in_specs, out_specs=out_specs,
        out_shape=out_shape)(*args)
    return res[0] if Wnext is None else (res[0], res[1])


def _final_tc(h, agg, deg, Wself, Wneigh, b):
    def kfn(h_ref, agg_ref, deg_ref, ws_ref, wn_ref, b_ref, o_ref):
        mean = _mean(agg_ref[...], deg_ref[...])
        o_ref[...] = jnp.dot(h_ref[...], ws_ref[...],
                             preferred_element_type=jnp.float32) \
            + jnp.dot(mean, wn_ref[...],
                      preferred_element_type=jnp.float32) \
            + b_ref[...]

    return pl.pallas_call(
        kfn, grid=(N // BR,),
        in_specs=[
            pl.BlockSpec((BR, D_H), lambda i: (i, 0)),
            pl.BlockSpec((NC, BR, D_H), lambda i: (0, i, 0)),
            pl.BlockSpec((NC, BR, D_H), lambda i: (0, i, 0)),
            pl.BlockSpec((D_H, D_OUT), lambda i: (0, 0)),
            pl.BlockSpec((D_H, D_OUT), lambda i: (0, 0)),
            pl.BlockSpec((1, D_OUT), lambda i: (0, 0)),
        ],
        out_specs=pl.BlockSpec((BR, D_OUT), lambda i: (i, 0)),
        out_shape=jax.ShapeDtypeStruct((N, D_OUT), jnp.float32),
    )(h, agg, deg, Wself, Wneigh, b.reshape(1, D_OUT))


def kernel(x, edge_index, Wself0, Wneigh0, b0, Wself1, Wneigh1, b1,
           Wself2, Wneigh2, b2):
    src = edge_index[0]
    dst = edge_index[1]
    pad = E_PAD - E
    src3 = jnp.concatenate(
        [src, jnp.zeros((pad,), jnp.int32)]).reshape(NW, NCHUNK, CHUNK)
    dst3 = jnp.concatenate(
        [dst, jnp.full((pad,), N, jnp.int32)]).reshape(NW, NCHUNK, CHUNK)
    zeros128 = jnp.zeros((N_PAD, D_H), jnp.float32)
    ones128 = jnp.ones((CHUNK, D_H), jnp.float32)

    deg = _deg_kernel()(dst3, zeros128, ones128)
    agg0 = _seg_sum_kernel()(jnp.stack([x, x]), src3, dst3, zeros128)
    h1 = _layer_tc(x, agg0, deg, Wself0, Wneigh0, b0)
    agg1 = _seg_sum_kernel()(jnp.stack([h1, h1]), src3, dst3, zeros128)
    h2 = _layer_tc(h1, agg1, deg, Wself1, Wneigh1, b1)
    agg2 = _seg_sum_kernel()(jnp.stack([h2, h2]), src3, dst3, zeros128)
    return _final_tc(h2, agg2, deg, Wself2, Wneigh2, b2)
